# Initial kernel scaffold; baseline (speedup 1.0000x reference)
import jax
import jax.numpy as jnp
from jax.experimental import pallas as pl


def kernel(batch_size, row_embed, col_embed):
    raise NotImplementedError("write your pallas kernel here")


# speedup vs baseline: 1.2056x; 1.2056x over previous
import jax
import jax.numpy as jnp
from jax.experimental import pallas as pl

HEIGHT, WIDTH, DIM, BATCH = 32, 32, 768, 64


def _pos_kernel(row_ref, col_ref, out_ref):
    r = row_ref[:]
    c = col_ref[:]
    pos = (r[:, None, :] + c[None, :, :]).reshape(HEIGHT * WIDTH, DIM)
    out_ref[0] = pos


def kernel(batch_size, row_embed, col_embed):
    del batch_size
    out = pl.pallas_call(
        _pos_kernel,
        grid=(BATCH,),
        in_specs=[
            pl.BlockSpec((HEIGHT, DIM), lambda b: (0, 0)),
            pl.BlockSpec((WIDTH, DIM), lambda b: (0, 0)),
        ],
        out_specs=pl.BlockSpec((1, HEIGHT * WIDTH, DIM), lambda b: (b, 0, 0)),
        out_shape=jax.ShapeDtypeStruct((BATCH, HEIGHT * WIDTH, DIM), jnp.float32),
    )(row_embed, col_embed)
    return out
